# Initial kernel scaffold; baseline (speedup 1.0000x reference)
import jax
import jax.numpy as jnp
from jax.experimental import pallas as pl


def kernel(x, eye):
    raise NotImplementedError("write your pallas kernel here")


# speedup vs baseline: 1.1992x; 1.1992x over previous
import functools

import jax
import jax.numpy as jnp
from jax import lax
from jax.experimental import pallas as pl
from jax.experimental.pallas import tpu as pltpu
from jax.experimental.pallas import tpu_sc as plsc

N_CAT = 1000
L = 16
NC = 2
NS = 16
NW = NC * NS
CHUNK = 32


def _one_hot_sc(x_flat, n_rows):
    b_per_w = n_rows // NW
    n_chunks = b_per_w // CHUNK
    mesh = plsc.VectorSubcoreMesh(core_axis_name="c", subcore_axis_name="s")

    @functools.partial(
        pl.kernel,
        out_type=jax.ShapeDtypeStruct((n_rows * N_CAT,), jnp.float32),
        mesh=mesh,
        scratch_types=[
            pltpu.VMEM((b_per_w,), jnp.int32),
            pltpu.VMEM((CHUNK * N_CAT,), jnp.float32),
        ],
        compiler_params=pltpu.CompilerParams(needs_layout_passes=False),
    )
    def body(x_hbm, out_hbm, idx_v, buf_v):
        wid = lax.axis_index("s") * NC + lax.axis_index("c")
        base = wid * b_per_w

        pltpu.sync_copy(x_hbm.at[pl.ds(base, b_per_w)], idx_v)

        zeros = jnp.zeros((L,), jnp.float32)
        ones = jnp.ones((L,), jnp.float32)
        lane = lax.iota(jnp.int32, L)

        def zero_body(i, _):
            buf_v[pl.ds(i * L, L)] = zeros
            return 0

        lax.fori_loop(0, (CHUNK * N_CAT) // L, zero_body, 0)

        def scatter_vals(row0, vals):
            for g in range(CHUNK // L):
                cols = idx_v[pl.ds(row0 + g * L, L)]
                pos = (g * L + lane) * N_CAT + cols
                plsc.store_scatter(buf_v, [pos], vals)

        def chunk_body(k, _):
            row0 = k * CHUNK
            scatter_vals(row0, ones)
            pltpu.sync_copy(
                buf_v, out_hbm.at[pl.ds((base + row0) * N_CAT, CHUNK * N_CAT)]
            )
            scatter_vals(row0, zeros)
            return 0

        lax.fori_loop(0, n_chunks, chunk_body, 0)

    return body(x_flat)


def kernel(x, eye):
    n_rows = x.shape[0] * x.shape[1]
    x_flat = x.reshape(n_rows).astype(jnp.int32)
    out_flat = _one_hot_sc(x_flat, n_rows)
    return out_flat.reshape(x.shape[0], x.shape[1], N_CAT)
